# Initial kernel scaffold; baseline (speedup 1.0000x reference)
import jax
import jax.numpy as jnp
from jax.experimental import pallas as pl


def kernel(seq, adj, W, bias, prelu_a):
    raise NotImplementedError("write your pallas kernel here")


# speedup vs baseline: 1.0405x; 1.0405x over previous
import jax
import jax.numpy as jnp
from jax import lax
from jax.experimental import pallas as pl
from jax.experimental.pallas import tpu as pltpu


def _gcn_kernel(seq_ref, w_ref, adj_ref, bias_ref, a_ref, out_ref, fts_ref):
    @pl.when(pl.program_id(0) == 0)
    def _():
        fts_ref[...] = lax.dot_general(
            seq_ref[...], w_ref[...],
            dimension_numbers=(((1,), (1,)), ((), ())),
            preferred_element_type=jnp.float32,
        )

    y = jnp.dot(adj_ref[...], fts_ref[...], preferred_element_type=jnp.float32)
    y = y + bias_ref[...]
    a = a_ref[0, 0]
    out_ref[...] = jnp.maximum(y, 0.0) + a * jnp.minimum(y, 0.0)


def kernel(seq, adj, W, bias, prelu_a):
    _, n, in_ft = seq.shape
    out_ft = W.shape[0]
    tm = 400
    grid = (n // tm,)

    out = pl.pallas_call(
        _gcn_kernel,
        grid=grid,
        in_specs=[
            pl.BlockSpec((n, in_ft), lambda i: (0, 0)),
            pl.BlockSpec((out_ft, in_ft), lambda i: (0, 0)),
            pl.BlockSpec((tm, n), lambda i: (i, 0)),
            pl.BlockSpec((1, out_ft), lambda i: (0, 0)),
            pl.BlockSpec(memory_space=pltpu.SMEM),
        ],
        out_specs=pl.BlockSpec((tm, out_ft), lambda i: (i, 0)),
        out_shape=jax.ShapeDtypeStruct((n, out_ft), jnp.float32),
        scratch_shapes=[pltpu.VMEM((n, out_ft), jnp.float32)],
        compiler_params=pltpu.CompilerParams(
            dimension_semantics=("arbitrary",),
        ),
    )(seq[0], W, adj[0], bias.reshape(1, out_ft), prelu_a.reshape(1, 1))

    return out[None]
